# baseline (speedup 1.0000x reference)
import jax
import jax.numpy as jnp
from jax import lax
from jax.experimental import pallas as pl
from jax.experimental.pallas import tpu as pltpu
from jax.experimental.pallas import tpu_sc as plsc

BATCH = 16384
HIST = 200
N = BATCH * HIST
VOCAB = 1000000
PAD_VOCAB = 1024000
WORDS = PAD_VOCAB // 32

_info = plsc.get_sparse_core_info()
_NC, _NS = _info.num_cores, _info.num_subcores


def _pack_body(x_ref, o_ref):
    xi = x_ref[...].astype(jnp.int32)
    shifts = lax.broadcasted_iota(jnp.int32, (WORDS, 32), 1)
    o_ref[...] = jnp.sum(jnp.left_shift(xi, shifts), axis=1)


def _lookup_body(idx_hbm, packed_hbm, out_hbm, packed_v,
                 idx_a, idx_b, out_a, out_b,
                 sem_ia, sem_ib, sem_oa, sem_ob):
    cid = lax.axis_index("c")
    sid = lax.axis_index("s")
    w = sid * _NC + cid
    pltpu.sync_copy(packed_hbm, packed_v)

    hs = w // 4
    tcb = w % 4
    ibufs = (idx_a, idx_b)
    isems = (sem_ia, sem_ib)
    obufs = (out_a, out_b)
    osems = (sem_oa, sem_ob)

    def start_in(tr):
        return pltpu.async_copy(
            idx_hbm.at[tr, pl.ds(tcb * 32, 32), pl.ds(hs * 128, 128)],
            ibufs[tr % 2], isems[tr % 2])

    hin = [None] * 25
    hout = [None] * 25
    hin[0] = start_in(0)
    for tr in range(25):
        if tr + 1 < 25:
            hin[tr + 1] = start_in(tr + 1)
        hin[tr].wait()
        if tr >= 2:
            hout[tr - 2].wait()
        idx_v = ibufs[tr % 2]
        out_v = obufs[tr % 2]

        @plsc.parallel_loop(0, 256, unroll=8)
        def _(g):
            j = lax.shift_right_logical(g, 3)
            c16 = jnp.bitwise_and(g, 7) * 16
            iv = idx_v[j, pl.ds(c16, 16)]
            wi = lax.shift_right_logical(iv, 5)
            bi = jnp.bitwise_and(iv, 31)
            wd = plsc.load_gather(packed_v, [wi])
            bit = jnp.bitwise_and(lax.shift_right_logical(wd, bi), 1)
            out_v[pl.ds(g * 16, 16)] = bit.astype(jnp.float32)

        hout[tr] = pltpu.async_copy(
            out_v, out_hbm.at[tr * 8 + hs, pl.ds(tcb * 4096, 4096)],
            osems[tr % 2])
    hout[23].wait()
    hout[24].wait()


@jax.jit
def _run(idx3, tbl_flat):
    packed = pl.pallas_call(
        _pack_body,
        out_shape=jax.ShapeDtypeStruct((WORDS,), jnp.int32),
    )(tbl_flat.reshape(WORDS, 32))

    mesh = plsc.VectorSubcoreMesh(core_axis_name="c", subcore_axis_name="s")
    return pl.kernel(
        _lookup_body,
        mesh=mesh,
        compiler_params=pltpu.CompilerParams(
            needs_layout_passes=False, use_tc_tiling_on_sc=False
        ),
        out_type=jax.ShapeDtypeStruct((HIST, BATCH), jnp.float32),
        scratch_types=[
            pltpu.VMEM((WORDS,), jnp.int32),
            pltpu.VMEM((32, 128), jnp.int32),
            pltpu.VMEM((32, 128), jnp.int32),
            pltpu.VMEM((4096,), jnp.float32),
            pltpu.VMEM((4096,), jnp.float32),
            pltpu.SemaphoreType.DMA,
            pltpu.SemaphoreType.DMA,
            pltpu.SemaphoreType.DMA,
            pltpu.SemaphoreType.DMA,
        ],
    )(idx3, packed)


def kernel(input, table):
    idx3 = input.T.reshape(25, 8, 128, 128).transpose(0, 2, 1, 3).reshape(
        25, 128, 1024
    )
    tbl_flat = jnp.pad(table.reshape(-1), (0, PAD_VOCAB - VOCAB))
    out = _run(idx3, tbl_flat)
    return out.T.reshape(BATCH, HIST, 1)


# speedup vs baseline: 1.1969x; 1.1969x over previous
import jax
import jax.numpy as jnp
from jax import lax
from jax.experimental import pallas as pl
from jax.experimental.pallas import tpu as pltpu
from jax.experimental.pallas import tpu_sc as plsc

BATCH = 16384
HIST = 200
N = BATCH * HIST
VOCAB = 1000000
WORDS = 32000
WORDS_PER_TILE = WORDS // 16
ROUND = 12800
RG = ROUND // 512
SKEW = ROUND + ROUND // 32

_info = plsc.get_sparse_core_info()
_NC, _NS = _info.num_cores, _info.num_subcores


def _pack_chunk(tbl_flat, start, n, stage, skew_v, slice_v, dst_off,
                consts, masked):
    iota1, iota32, iota33, zeros = consts
    pltpu.sync_copy(tbl_flat.at[0, pl.ds(start, n)], stage.at[pl.ds(0, n)])

    @plsc.parallel_loop(0, n // 16, unroll=4)
    def _(q):
        p = q * 16
        v = stage[pl.ds(p, 16)]
        skew_v[pl.ds(p + lax.shift_right_logical(p, 5), 16)] = v

    @plsc.parallel_loop(0, n // 512, unroll=2)
    def _(g):
        acc = jnp.zeros((16,), jnp.int32)
        for b in range(32):
            idx = iota33 + (g * 528 + b)
            v = plsc.bitcast(plsc.load_gather(skew_v, [idx]), jnp.int32)
            if b <= 23:
                bits = lax.shift_right_logical(v, 23 - b)
            else:
                bits = lax.shift_left(v, b - 23)
            mask_b = jnp.int32(-(2**31)) if b == 31 else jnp.int32(1 << b)
            bits = jnp.bitwise_and(bits, mask_b)
            if masked:
                e = iota32 + (g * 512 + b)
                bits = jnp.where(e < 1600, bits, 0)
            acc = jnp.bitwise_or(acc, bits)
        slice_v[pl.ds(dst_off + g * 16, 16)] = acc


def _body(idx_hbm, tbl_hbm, out_hbm, xchg_hbm, packed_v, slice_v, stage,
          skew_v, idx_a, idx_b, out_a, out_b, sem_ia, sem_ib, sem_oa,
          sem_ob):
    cid = lax.axis_index("c")
    sid = lax.axis_index("s")
    w = sid * _NC + cid
    iota1 = lax.iota(jnp.int32, 16)
    consts = (iota1, iota1 * 32, iota1 * 33, jnp.zeros((16,), jnp.int32))
    tbl_flat = tbl_hbm

    with jax.named_scope("pack"):
        base = sid * (WORDS_PER_TILE * 32)
        for r in range(5):
            start = base + r * ROUND
            dst = r * (ROUND // 32)
            if r <= 2:
                _pack_chunk(tbl_flat, start, ROUND, stage, skew_v,
                            slice_v, dst, consts, False)
            elif r == 3:
                @pl.when(sid < 15)
                def _():
                    _pack_chunk(tbl_flat, start, ROUND, stage, skew_v,
                                slice_v, dst, consts, False)

                @pl.when(sid == 15)
                def _():
                    _pack_chunk(tbl_flat, VOCAB - 1600, 1600, stage,
                                skew_v, slice_v, dst, consts, True)
            else:
                @pl.when(sid < 15)
                def _():
                    _pack_chunk(tbl_flat, start, ROUND, stage, skew_v,
                                slice_v, dst, consts, False)

                @pl.when(sid == 15)
                def _():
                    @plsc.parallel_loop(0, RG)
                    def _(g):
                        slice_v[pl.ds(dst + g * 16, 16)] = jnp.zeros(
                            (16,), jnp.int32)

    with jax.named_scope("xchg"):
        pltpu.sync_copy(slice_v,
                        xchg_hbm.at[cid, pl.ds(sid * WORDS_PER_TILE,
                                               WORDS_PER_TILE)])
        plsc.subcore_barrier()
        pltpu.sync_copy(xchg_hbm.at[cid], packed_v)

    scope_b = jax.named_scope("lookup")
    scope_b.__enter__()
    hs = w // 4
    tcb = w % 4
    ibufs = (idx_a, idx_b)
    isems = (sem_ia, sem_ib)
    obufs = (out_a, out_b)
    osems = (sem_oa, sem_ob)

    def start_in(tr):
        return pltpu.async_copy(
            idx_hbm.at[tr, pl.ds(tcb * 32, 32), pl.ds(hs * 128, 128)],
            ibufs[tr % 2], isems[tr % 2])

    hin = [None] * 25
    hout = [None] * 25
    hin[0] = start_in(0)
    for tr in range(25):
        if tr + 1 < 25:
            hin[tr + 1] = start_in(tr + 1)
        hin[tr].wait()
        if tr >= 2:
            hout[tr - 2].wait()
        idx_v = ibufs[tr % 2]
        out_v = obufs[tr % 2]

        @plsc.parallel_loop(0, 256, unroll=8)
        def _(g):
            j = lax.shift_right_logical(g, 3)
            c16 = jnp.bitwise_and(g, 7) * 16
            iv = idx_v[j, pl.ds(c16, 16)]
            wi = lax.shift_right_logical(iv, 5)
            bi = jnp.bitwise_and(iv, 31)
            wd = plsc.load_gather(packed_v, [wi])
            bit = jnp.bitwise_and(lax.shift_right_logical(wd, bi), 1)
            out_v[pl.ds(g * 16, 16)] = bit.astype(jnp.float32)

        hout[tr] = pltpu.async_copy(
            out_v, out_hbm.at[tr * 8 + hs, pl.ds(tcb * 4096, 4096)],
            osems[tr % 2])
    hout[23].wait()
    hout[24].wait()
    scope_b.__exit__(None, None, None)


@jax.jit
def _run(idx3, table):
    mesh = plsc.VectorSubcoreMesh(core_axis_name="c", subcore_axis_name="s")
    out, _ = pl.kernel(
        _body,
        mesh=mesh,
        compiler_params=pltpu.CompilerParams(
            needs_layout_passes=False, use_tc_tiling_on_sc=False
        ),
        out_type=(
            jax.ShapeDtypeStruct((HIST, BATCH), jnp.float32),
            jax.ShapeDtypeStruct((_NC, WORDS), jnp.int32),
        ),
        scratch_types=[
            pltpu.VMEM((WORDS,), jnp.int32),
            pltpu.VMEM((WORDS_PER_TILE,), jnp.int32),
            pltpu.VMEM((ROUND,), jnp.float32),
            pltpu.VMEM((SKEW,), jnp.float32),
            pltpu.VMEM((32, 128), jnp.int32),
            pltpu.VMEM((32, 128), jnp.int32),
            pltpu.VMEM((4096,), jnp.float32),
            pltpu.VMEM((4096,), jnp.float32),
            pltpu.SemaphoreType.DMA,
            pltpu.SemaphoreType.DMA,
            pltpu.SemaphoreType.DMA,
            pltpu.SemaphoreType.DMA,
        ],
    )(idx3, table)
    return out


def kernel(input, table):
    idx3 = input.T.reshape(25, 8, 128, 128).transpose(0, 2, 1, 3).reshape(
        25, 128, 1024
    )
    out = _run(idx3, table.T)
    return out.T.reshape(BATCH, HIST, 1)
